# baseline (speedup 1.0000x reference)
import functools

import jax
import jax.numpy as jnp
from jax.experimental import pallas as pl
from jax.experimental.pallas import tpu as pltpu

WS = 7
TP = WS * 8
TOPK = 4
NUM_HEADS = 8


def _qkv_kernel(x_ref, wq_ref, wk_ref, wv_ref, bq_ref, bk_ref, bv_ref,
                q_ref, kv_ref, xr_ref):
    f32 = jnp.float32
    xb = x_ref[...].transpose(0, 2, 1)
    C = xb.shape[-1]
    nW = xb.shape[1] // WS
    x4 = xb.reshape(WS, nW, WS, C)
    xr_ref[...] = jnp.mean(x4, axis=(0, 2))
    x4p = jnp.pad(x4, ((0, 0), (0, 0), (0, 1), (0, 0)))
    xw = x4p.transpose(1, 0, 2, 3).reshape(nW * TP, C)
    q = jnp.dot(xw, wq_ref[...], preferred_element_type=f32) + bq_ref[...]
    k = jnp.dot(xw, wk_ref[...], preferred_element_type=f32) + bk_ref[...]
    v = jnp.dot(xw, wv_ref[...], preferred_element_type=f32) + bv_ref[...]
    bf16 = jnp.bfloat16
    q_ref[...] = q.reshape(nW, TP, C).astype(bf16)
    kv_ref[...] = jnp.stack(
        [k.reshape(nW, TP, C), v.reshape(nW, TP, C)], axis=1).astype(bf16)


def _routing_kernel(xr_ref, wq_ref, wk_ref, bq_ref, bk_ref,
                    g1_ref, gb1_ref, g2_ref, gb2_ref,
                    idx_ref, gwt_ref, *, scale, n_grid):
    f32 = jnp.float32
    xr = xr_ref[...]
    S = xr.shape[0]
    q_r = jnp.dot(xr, wq_ref[...], preferred_element_type=f32) + bq_ref[...]
    k_r = jnp.dot(xr, wk_ref[...], preferred_element_type=f32) + bk_ref[...]
    scores = jax.lax.dot_general(
        q_r, k_r, (((1,), (1,)), ((), ())), preferred_element_type=f32) * scale

    col = jax.lax.broadcasted_iota(jnp.int32, (S, S), 1)
    neg_inf = jnp.float32(-jnp.inf)
    idx_cols = []
    a = scores
    for _ in range(TOPK):
        m = jnp.max(a, axis=1, keepdims=True)
        hit = a >= m
        idx_t = jnp.min(jnp.where(hit, col, S), axis=1)
        idx_cols.append(idx_t)
        a = jnp.where(col == idx_t[:, None], neg_inf, a)

    h = jnp.maximum(jnp.dot(q_r, g1_ref[...], preferred_element_type=f32)
                    + gb1_ref[...], 0.0)
    gp = jnp.dot(h, g2_ref[...], preferred_element_type=f32) + gb2_ref[...]
    sp = jnp.maximum(gp, 0.0) + jnp.log1p(jnp.exp(-jnp.abs(gp)))
    sx = sp[:, 0:1] + 0.5
    sy = sp[:, 1:2] + 0.5

    row = jax.lax.broadcasted_iota(jnp.int32, (S, 1), 0)
    ix = (row % n_grid).astype(f32)
    iy = (row // n_grid).astype(f32)
    jgrid = jax.lax.broadcasted_iota(jnp.int32, (S, n_grid), 1).astype(f32)
    ex = jnp.exp(-0.5 * ((jgrid - ix) / sx) ** 2)
    ey = jnp.exp(-0.5 * ((jgrid - iy) / sy) ** 2)
    denom = (jnp.sum(ex, axis=1, keepdims=True)
             * jnp.sum(ey, axis=1, keepdims=True)) + 1e-6

    w_cols = []
    for t in range(TOPK):
        it = idx_cols[t][:, None]
        tx = (it % n_grid).astype(f32)
        ty = (it // n_grid).astype(f32)
        g = jnp.exp(-0.5 * (((tx - ix) / sx) ** 2 + ((ty - iy) / sy) ** 2))
        w_cols.append(g / denom)

    idx_ref[...] = jnp.concatenate([c[:, None] for c in idx_cols], axis=1)
    gwt_ref[...] = jnp.concatenate(w_cols, axis=1)


def _attn_kernel(idx_ref, gwt_ref, q_ref, *rest, scale, rb):
    f32 = jnp.float32
    kv_refs = rest[:rb * TOPK]
    own_refs = rest[rb * TOPK:rb * TOPK + rb]
    dw_ref, dwb_ref, pw_ref, pb_ref = rest[rb * TOPK + rb:rb * TOPK + rb + 4]
    o_ref = rest[rb * TOPK + rb + 4]
    i = pl.program_id(0)
    bf16 = jnp.bfloat16
    C = q_ref.shape[-1]
    lane = jax.lax.broadcasted_iota(jnp.int32, (1, TP), 1)
    colbias = jnp.where(lane % 8 == WS, jnp.float32(-jnp.inf), 0.0)
    outs = []
    for r in range(rb):
        q = q_ref[r]
        base = (i * rb + r) * TOPK
        ws_ = [gwt_ref[base + j] for j in range(TOPK)]
        sjs = [
            jax.lax.dot_general(
                q, kv_refs[r * TOPK + j][0, 0], (((1,), (1,)), ((), ())),
                preferred_element_type=f32) * (ws_[j] * scale) + colbias
            for j in range(TOPK)
        ]
        m = jnp.maximum(
            jnp.maximum(jnp.max(sjs[0], axis=1, keepdims=True),
                        jnp.max(sjs[1], axis=1, keepdims=True)),
            jnp.maximum(jnp.max(sjs[2], axis=1, keepdims=True),
                        jnp.max(sjs[3], axis=1, keepdims=True)))
        es = [jnp.exp(sj - m) for sj in sjs]
        denom = ((jnp.sum(es[0], axis=1, keepdims=True)
                  + jnp.sum(es[1], axis=1, keepdims=True))
                 + (jnp.sum(es[2], axis=1, keepdims=True)
                    + jnp.sum(es[3], axis=1, keepdims=True)))
        a01 = jax.lax.dot_general(
            (es[0] * ws_[0]).astype(bf16), kv_refs[r * TOPK][0, 1],
            (((1,), (0,)), ((), ())), preferred_element_type=f32)
        a01 += jax.lax.dot_general(
            (es[1] * ws_[1]).astype(bf16), kv_refs[r * TOPK + 1][0, 1],
            (((1,), (0,)), ((), ())), preferred_element_type=f32)
        a23 = jax.lax.dot_general(
            (es[2] * ws_[2]).astype(bf16), kv_refs[r * TOPK + 2][0, 1],
            (((1,), (0,)), ((), ())), preferred_element_type=f32)
        a23 += jax.lax.dot_general(
            (es[3] * ws_[3]).astype(bf16), kv_refs[r * TOPK + 3][0, 1],
            (((1,), (0,)), ((), ())), preferred_element_type=f32)
        v4 = own_refs[r][0, 1].astype(f32).reshape(WS, 8, C)
        colv = jax.lax.broadcasted_iota(jnp.int32, (1, 8, 1), 1)
        v4 = jnp.where(colv < WS, v4, 0.0)
        vp = jnp.pad(v4, ((1, 1), (1, 1), (0, 0)))
        acc = dwb_ref[...][None]
        for ky in range(3):
            for kx in range(3):
                acc = acc + vp[ky:ky + WS, kx:kx + 8, :] * dw_ref[ky, kx][None, None, :]
        outs.append((a01 + a23) * (1.0 / denom) + acc.reshape(TP, C))
    o = jnp.concatenate(outs, axis=0)
    o = jnp.dot(o, pw_ref[...], preferred_element_type=f32) + pb_ref[...]
    o = o.reshape(rb, WS, 8, C).transpose(1, 0, 2, 3)[:, :, :WS, :]
    o_ref[...] = o.reshape(WS, rb * WS, C).transpose(0, 2, 1)


def kernel(x, qkv_w, qkv_b, gp_w1, gp_b1, gp_w2, gp_b2, dw_w, dw_b,
           proj_w, proj_b):
    B, H, W, C = x.shape
    ws = WS
    nH, nW = H // ws, W // ws
    S = nH * nW
    scale = float(C // NUM_HEADS) ** (-0.5)
    f32 = jnp.float32

    x3 = jnp.transpose(x[0], (0, 2, 1))
    wq = qkv_w[0:C].T
    wk = qkv_w[C:2 * C].T
    wv = qkv_w[2 * C:3 * C].T
    bq = qkv_b[0:C][None]
    bk = qkv_b[C:2 * C][None]
    bv = qkv_b[2 * C:3 * C][None]
    dw2 = dw_w[:, 0].transpose(1, 2, 0)
    dwb2 = dw_b[None]

    full2 = lambda a: pl.BlockSpec(a.shape, lambda i: (0,) * a.ndim)
    q, kv, xr = pl.pallas_call(
        _qkv_kernel,
        grid=(nH,),
        in_specs=[
            pl.BlockSpec((ws, C, W), lambda i: (i, 0, 0)),
            full2(wq), full2(wk), full2(wv), full2(bq), full2(bk), full2(bv),
        ],
        out_specs=[
            pl.BlockSpec((nW, TP, C), lambda i: (i, 0, 0)),
            pl.BlockSpec((nW, 2, TP, C), lambda i: (i, 0, 0, 0)),
            pl.BlockSpec((nW, C), lambda i: (i, 0)),
        ],
        out_shape=[
            jax.ShapeDtypeStruct((S, TP, C), jnp.bfloat16),
            jax.ShapeDtypeStruct((S, 2, TP, C), jnp.bfloat16),
            jax.ShapeDtypeStruct((S, C), f32),
        ],
    )(x3, wq, wk, wv, bq, bk, bv)

    g1 = gp_w1.T
    gb1 = gp_b1[None]
    g2 = gp_w2.T
    gb2 = gp_b2[None]
    fullb = lambda a: pl.BlockSpec(a.shape, lambda i: (0,) * a.ndim)
    idx, gwt = pl.pallas_call(
        functools.partial(_routing_kernel, scale=scale, n_grid=nW),
        grid=(1,),
        in_specs=[fullb(xr), fullb(wq), fullb(wk), fullb(bq), fullb(bk),
                  fullb(g1), fullb(gb1), fullb(g2), fullb(gb2)],
        out_specs=[fullb(jnp.empty((S, TOPK), jnp.int32)),
                   fullb(jnp.empty((S, TOPK), f32))],
        out_shape=[jax.ShapeDtypeStruct((S, TOPK), jnp.int32),
                   jax.ShapeDtypeStruct((S, TOPK), f32)],
    )(xr, wq, wk, bq, bk, g1, gb1, g2, gb2)

    pw = proj_w.T
    pb = proj_b[None]

    RB = nW
    own = pl.BlockSpec((RB, TP, C), lambda i, idx_r, gwt_r: (i, 0, 0))

    def gspec(r, j):
        return pl.BlockSpec(
            (1, 2, TP, C),
            lambda i, idx_r, gwt_r: (idx_r[(i * RB + r) * TOPK + j], 0, 0, 0))

    def ospec(r):
        return pl.BlockSpec(
            (1, 2, TP, C), lambda i, idx_r, gwt_r: (i * RB + r, 0, 0, 0))

    fullp = lambda a: pl.BlockSpec(
        a.shape, lambda i, idx_r, gwt_r: (0,) * a.ndim)

    out_t = pl.pallas_call(
        functools.partial(_attn_kernel, scale=scale, rb=RB),
        grid_spec=pltpu.PrefetchScalarGridSpec(
            num_scalar_prefetch=2,
            grid=(S // RB,),
            in_specs=[own]
            + [gspec(r, j) for r in range(RB) for j in range(TOPK)]
            + [ospec(r) for r in range(RB)]
            + [fullp(dw2), fullp(dwb2), fullp(pw), fullp(pb)],
            out_specs=pl.BlockSpec(
                (ws, C, RB * ws),
                lambda i, idx_r, gwt_r: (i, 0, 0)),
        ),
        out_shape=jax.ShapeDtypeStruct((H, C, W), f32),
    )(idx.reshape(-1), gwt.reshape(-1), q,
      *([kv] * (RB * TOPK)), *([kv] * RB), dw2, dwb2, pw, pb)

    return jnp.transpose(out_t, (0, 2, 1)).reshape(1, H, W, C)


# speedup vs baseline: 1.0947x; 1.0947x over previous
import functools

import jax
import jax.numpy as jnp
from jax.experimental import pallas as pl
from jax.experimental.pallas import tpu as pltpu

WS = 7
TP = WS * 8
TOPK = 4
NUM_HEADS = 8


def _qkv_kernel(x_ref, wq_ref, wk_ref, wv_ref, bq_ref, bk_ref, bv_ref,
                dw_ref, dwb_ref,
                q_ref, kv_ref, vdw_ref, xr_ref):
    f32 = jnp.float32
    xb = x_ref[...].transpose(0, 2, 1)
    C = xb.shape[-1]
    nW = xb.shape[1] // WS
    x4 = xb.reshape(WS, nW, WS, C)
    xr_ref[...] = jnp.mean(x4, axis=(0, 2))
    x4p = jnp.pad(x4, ((0, 0), (0, 0), (0, 1), (0, 0)))
    xw = x4p.transpose(1, 0, 2, 3).reshape(nW * TP, C)
    q = jnp.dot(xw, wq_ref[...], preferred_element_type=f32) + bq_ref[...]
    k = jnp.dot(xw, wk_ref[...], preferred_element_type=f32) + bk_ref[...]
    v = jnp.dot(xw, wv_ref[...], preferred_element_type=f32) + bv_ref[...]
    bf16 = jnp.bfloat16
    q_ref[...] = q.reshape(nW, TP, C).astype(bf16)
    kv_ref[...] = jnp.stack(
        [k.reshape(nW, TP, C), v.reshape(nW, TP, C)], axis=1).astype(bf16)

    v4 = v.reshape(nW, WS, 8, C)
    col = jax.lax.broadcasted_iota(jnp.int32, (1, 1, 8, 1), 2)
    v4 = jnp.where(col < WS, v4, 0.0)
    vp = jnp.pad(v4, ((0, 0), (1, 1), (1, 1), (0, 0)))
    cols = [vp[:, :, kx:kx + 8, :] for kx in range(3)]
    acc = jnp.zeros((nW, WS, 8, C), f32)
    for ky in range(3):
        for kx in range(3):
            acc = acc + cols[kx][:, ky:ky + WS] * dw_ref[ky, kx][None, None, None, :]
    vdw_ref[...] = (acc.reshape(nW, TP, C) + dwb_ref[...][None]).astype(bf16)


def _routing_kernel(xr_ref, wq_ref, wk_ref, bq_ref, bk_ref,
                    g1_ref, gb1_ref, g2_ref, gb2_ref,
                    idx_ref, gwt_ref, *, scale, n_grid):
    f32 = jnp.float32
    xr = xr_ref[...]
    S = xr.shape[0]
    q_r = jnp.dot(xr, wq_ref[...], preferred_element_type=f32) + bq_ref[...]
    k_r = jnp.dot(xr, wk_ref[...], preferred_element_type=f32) + bk_ref[...]
    scores = jax.lax.dot_general(
        q_r, k_r, (((1,), (1,)), ((), ())), preferred_element_type=f32) * scale

    col = jax.lax.broadcasted_iota(jnp.int32, (S, S), 1)
    neg_inf = jnp.float32(-jnp.inf)
    idx_cols = []
    a = scores
    for _ in range(TOPK):
        m = jnp.max(a, axis=1, keepdims=True)
        hit = a >= m
        idx_t = jnp.min(jnp.where(hit, col, S), axis=1)
        idx_cols.append(idx_t)
        a = jnp.where(col == idx_t[:, None], neg_inf, a)

    h = jnp.maximum(jnp.dot(q_r, g1_ref[...], preferred_element_type=f32)
                    + gb1_ref[...], 0.0)
    gp = jnp.dot(h, g2_ref[...], preferred_element_type=f32) + gb2_ref[...]
    sp = jnp.maximum(gp, 0.0) + jnp.log1p(jnp.exp(-jnp.abs(gp)))
    sx = sp[:, 0:1] + 0.5
    sy = sp[:, 1:2] + 0.5

    row = jax.lax.broadcasted_iota(jnp.int32, (S, 1), 0)
    ix = (row % n_grid).astype(f32)
    iy = (row // n_grid).astype(f32)
    jgrid = jax.lax.broadcasted_iota(jnp.int32, (S, n_grid), 1).astype(f32)
    ex = jnp.exp(-0.5 * ((jgrid - ix) / sx) ** 2)
    ey = jnp.exp(-0.5 * ((jgrid - iy) / sy) ** 2)
    denom = (jnp.sum(ex, axis=1, keepdims=True)
             * jnp.sum(ey, axis=1, keepdims=True)) + 1e-6

    w_cols = []
    for t in range(TOPK):
        it = idx_cols[t][:, None]
        tx = (it % n_grid).astype(f32)
        ty = (it // n_grid).astype(f32)
        g = jnp.exp(-0.5 * (((tx - ix) / sx) ** 2 + ((ty - iy) / sy) ** 2))
        w_cols.append(g / denom)

    idx_ref[...] = jnp.concatenate([c[:, None] for c in idx_cols], axis=1)
    gwt_ref[...] = jnp.concatenate(w_cols, axis=1)


def _attn_kernel(idx_ref, gwt_ref, q_ref, vdw_ref, *rest, scale, rb):
    f32 = jnp.float32
    kv_refs = rest[:rb * TOPK]
    pw_ref, pb_ref = rest[rb * TOPK], rest[rb * TOPK + 1]
    o_ref = rest[rb * TOPK + 2]
    i = pl.program_id(0)
    bf16 = jnp.bfloat16
    C = q_ref.shape[-1]
    lane = jax.lax.broadcasted_iota(jnp.int32, (1, TP), 1)
    colbias = jnp.where(lane % 8 == WS, jnp.float32(-jnp.inf), 0.0)
    outs = []
    for r in range(rb):
        q = q_ref[r]
        base = (i * rb + r) * TOPK
        ws_ = [gwt_ref[base + j] for j in range(TOPK)]
        sjs = [
            jax.lax.dot_general(
                q, kv_refs[r * TOPK + j][0, 0], (((1,), (1,)), ((), ())),
                preferred_element_type=f32) * (ws_[j] * scale) + colbias
            for j in range(TOPK)
        ]
        m = jnp.maximum(
            jnp.maximum(jnp.max(sjs[0], axis=1, keepdims=True),
                        jnp.max(sjs[1], axis=1, keepdims=True)),
            jnp.maximum(jnp.max(sjs[2], axis=1, keepdims=True),
                        jnp.max(sjs[3], axis=1, keepdims=True)))
        es = [jnp.exp(sj - m) for sj in sjs]
        denom = ((jnp.sum(es[0], axis=1, keepdims=True)
                  + jnp.sum(es[1], axis=1, keepdims=True))
                 + (jnp.sum(es[2], axis=1, keepdims=True)
                    + jnp.sum(es[3], axis=1, keepdims=True)))
        a01 = jax.lax.dot_general(
            (es[0] * ws_[0]).astype(bf16), kv_refs[r * TOPK][0, 1],
            (((1,), (0,)), ((), ())), preferred_element_type=f32)
        a01 += jax.lax.dot_general(
            (es[1] * ws_[1]).astype(bf16), kv_refs[r * TOPK + 1][0, 1],
            (((1,), (0,)), ((), ())), preferred_element_type=f32)
        a23 = jax.lax.dot_general(
            (es[2] * ws_[2]).astype(bf16), kv_refs[r * TOPK + 2][0, 1],
            (((1,), (0,)), ((), ())), preferred_element_type=f32)
        a23 += jax.lax.dot_general(
            (es[3] * ws_[3]).astype(bf16), kv_refs[r * TOPK + 3][0, 1],
            (((1,), (0,)), ((), ())), preferred_element_type=f32)
        outs.append((a01 + a23) * (1.0 / denom) + vdw_ref[r].astype(f32))
    o = jnp.concatenate(outs, axis=0)
    o = jnp.dot(o, pw_ref[...], preferred_element_type=f32) + pb_ref[...]
    o = o.reshape(rb, WS, 8, C).transpose(1, 0, 2, 3)[:, :, :WS, :]
    o_ref[...] = o.reshape(WS, rb * WS, C).transpose(0, 2, 1)


def kernel(x, qkv_w, qkv_b, gp_w1, gp_b1, gp_w2, gp_b2, dw_w, dw_b,
           proj_w, proj_b):
    B, H, W, C = x.shape
    ws = WS
    nH, nW = H // ws, W // ws
    S = nH * nW
    scale = float(C // NUM_HEADS) ** (-0.5)
    f32 = jnp.float32

    x3 = jnp.transpose(x[0], (0, 2, 1))
    wq = qkv_w[0:C].T
    wk = qkv_w[C:2 * C].T
    wv = qkv_w[2 * C:3 * C].T
    bq = qkv_b[0:C][None]
    bk = qkv_b[C:2 * C][None]
    bv = qkv_b[2 * C:3 * C][None]
    dw2 = dw_w[:, 0].transpose(1, 2, 0)
    dwb2 = dw_b[None]

    full2 = lambda a: pl.BlockSpec(a.shape, lambda i: (0,) * a.ndim)
    q, kv, vdw, xr = pl.pallas_call(
        _qkv_kernel,
        grid=(nH,),
        in_specs=[
            pl.BlockSpec((ws, C, W), lambda i: (i, 0, 0)),
            full2(wq), full2(wk), full2(wv), full2(bq), full2(bk), full2(bv),
            full2(dw2), full2(dwb2),
        ],
        out_specs=[
            pl.BlockSpec((nW, TP, C), lambda i: (i, 0, 0)),
            pl.BlockSpec((nW, 2, TP, C), lambda i: (i, 0, 0, 0)),
            pl.BlockSpec((nW, TP, C), lambda i: (i, 0, 0)),
            pl.BlockSpec((nW, C), lambda i: (i, 0)),
        ],
        out_shape=[
            jax.ShapeDtypeStruct((S, TP, C), jnp.bfloat16),
            jax.ShapeDtypeStruct((S, 2, TP, C), jnp.bfloat16),
            jax.ShapeDtypeStruct((S, TP, C), jnp.bfloat16),
            jax.ShapeDtypeStruct((S, C), f32),
        ],
    )(x3, wq, wk, wv, bq, bk, bv, dw2, dwb2)

    g1 = gp_w1.T
    gb1 = gp_b1[None]
    g2 = gp_w2.T
    gb2 = gp_b2[None]
    fullb = lambda a: pl.BlockSpec(a.shape, lambda i: (0,) * a.ndim)
    idx, gwt = pl.pallas_call(
        functools.partial(_routing_kernel, scale=scale, n_grid=nW),
        grid=(1,),
        in_specs=[fullb(xr), fullb(wq), fullb(wk), fullb(bq), fullb(bk),
                  fullb(g1), fullb(gb1), fullb(g2), fullb(gb2)],
        out_specs=[fullb(jnp.empty((S, TOPK), jnp.int32)),
                   fullb(jnp.empty((S, TOPK), f32))],
        out_shape=[jax.ShapeDtypeStruct((S, TOPK), jnp.int32),
                   jax.ShapeDtypeStruct((S, TOPK), f32)],
    )(xr, wq, wk, bq, bk, g1, gb1, g2, gb2)

    pw = proj_w.T
    pb = proj_b[None]

    RB = nW
    own = pl.BlockSpec((RB, TP, C), lambda i, idx_r, gwt_r: (i, 0, 0))

    def gspec(r, j):
        return pl.BlockSpec(
            (1, 2, TP, C),
            lambda i, idx_r, gwt_r: (idx_r[(i * RB + r) * TOPK + j], 0, 0, 0))

    fullp = lambda a: pl.BlockSpec(
        a.shape, lambda i, idx_r, gwt_r: (0,) * a.ndim)

    out_t = pl.pallas_call(
        functools.partial(_attn_kernel, scale=scale, rb=RB),
        grid_spec=pltpu.PrefetchScalarGridSpec(
            num_scalar_prefetch=2,
            grid=(S // RB,),
            in_specs=[own, own]
            + [gspec(r, j) for r in range(RB) for j in range(TOPK)]
            + [fullp(pw), fullp(pb)],
            out_specs=pl.BlockSpec(
                (ws, C, RB * ws),
                lambda i, idx_r, gwt_r: (i, 0, 0)),
        ),
        out_shape=jax.ShapeDtypeStruct((H, C, W), f32),
    )(idx.reshape(-1), gwt.reshape(-1), q, vdw,
      *([kv] * (RB * TOPK)), pw, pb)

    return jnp.transpose(out_t, (0, 2, 1)).reshape(1, H, W, C)


# speedup vs baseline: 1.0948x; 1.0000x over previous
import functools

import jax
import jax.numpy as jnp
from jax.experimental import pallas as pl
from jax.experimental.pallas import tpu as pltpu

WS = 7
TP = WS * 8
TOPK = 4
NUM_HEADS = 8


def _qkv_kernel(x_ref, wq_ref, wk_ref, wv_ref, bq_ref, bk_ref, bv_ref,
                dw_ref, dwb_ref, g1_ref, gb1_ref, g2_ref, gb2_ref,
                q_ref, kv_ref, vdw_ref, xr_ref, idx_ref, gwt_ref,
                *, scale, n_grid, n_steps):
    f32 = jnp.float32
    i = pl.program_id(0)
    xb = x_ref[...].transpose(0, 2, 1)
    C = xb.shape[-1]
    nW = xb.shape[1] // WS
    x4 = xb.reshape(WS, nW, WS, C)
    xr_ref[pl.ds(i * nW, nW), :] = jnp.mean(x4, axis=(0, 2))
    x4p = jnp.pad(x4, ((0, 0), (0, 0), (0, 1), (0, 0)))
    xw = x4p.transpose(1, 0, 2, 3).reshape(nW * TP, C)
    q = jnp.dot(xw, wq_ref[...], preferred_element_type=f32) + bq_ref[...]
    k = jnp.dot(xw, wk_ref[...], preferred_element_type=f32) + bk_ref[...]
    v = jnp.dot(xw, wv_ref[...], preferred_element_type=f32) + bv_ref[...]
    bf16 = jnp.bfloat16
    q_ref[...] = q.reshape(nW, TP, C).astype(bf16)
    kv_ref[...] = jnp.stack(
        [k.reshape(nW, TP, C), v.reshape(nW, TP, C)], axis=1).astype(bf16)

    v4 = v.reshape(nW, WS, 8, C)
    col = jax.lax.broadcasted_iota(jnp.int32, (1, 1, 8, 1), 2)
    v4 = jnp.where(col < WS, v4, 0.0)
    vp = jnp.pad(v4, ((0, 0), (1, 1), (1, 1), (0, 0)))
    cols = [vp[:, :, kx:kx + 8, :] for kx in range(3)]
    acc = jnp.zeros((nW, WS, 8, C), f32)
    for ky in range(3):
        for kx in range(3):
            acc = acc + cols[kx][:, ky:ky + WS] * dw_ref[ky, kx][None, None, None, :]
    vdw_ref[...] = (acc.reshape(nW, TP, C) + dwb_ref[...][None]).astype(bf16)

    @pl.when(i == n_steps - 1)
    def _():
        _routing_body(xr_ref, wq_ref, wk_ref, bq_ref, bk_ref,
                      g1_ref, gb1_ref, g2_ref, gb2_ref,
                      idx_ref, gwt_ref, scale=scale, n_grid=n_grid)


def _routing_body(xr_ref, wq_ref, wk_ref, bq_ref, bk_ref,
                  g1_ref, gb1_ref, g2_ref, gb2_ref,
                  idx_ref, gwt_ref, *, scale, n_grid):
    f32 = jnp.float32
    xr = xr_ref[...]
    S = xr.shape[0]
    q_r = jnp.dot(xr, wq_ref[...], preferred_element_type=f32) + bq_ref[...]
    k_r = jnp.dot(xr, wk_ref[...], preferred_element_type=f32) + bk_ref[...]
    scores = jax.lax.dot_general(
        q_r, k_r, (((1,), (1,)), ((), ())), preferred_element_type=f32) * scale

    col = jax.lax.broadcasted_iota(jnp.int32, (S, S), 1)
    neg_inf = jnp.float32(-jnp.inf)
    idx_cols = []
    a = scores
    for _ in range(TOPK):
        m = jnp.max(a, axis=1, keepdims=True)
        hit = a >= m
        idx_t = jnp.min(jnp.where(hit, col, S), axis=1)
        idx_cols.append(idx_t)
        a = jnp.where(col == idx_t[:, None], neg_inf, a)

    h = jnp.maximum(jnp.dot(q_r, g1_ref[...], preferred_element_type=f32)
                    + gb1_ref[...], 0.0)
    gp = jnp.dot(h, g2_ref[...], preferred_element_type=f32) + gb2_ref[...]
    sp = jnp.maximum(gp, 0.0) + jnp.log1p(jnp.exp(-jnp.abs(gp)))
    sx = sp[:, 0:1] + 0.5
    sy = sp[:, 1:2] + 0.5

    row = jax.lax.broadcasted_iota(jnp.int32, (S, 1), 0)
    ix = (row % n_grid).astype(f32)
    iy = (row // n_grid).astype(f32)
    jgrid = jax.lax.broadcasted_iota(jnp.int32, (S, n_grid), 1).astype(f32)
    ex = jnp.exp(-0.5 * ((jgrid - ix) / sx) ** 2)
    ey = jnp.exp(-0.5 * ((jgrid - iy) / sy) ** 2)
    denom = (jnp.sum(ex, axis=1, keepdims=True)
             * jnp.sum(ey, axis=1, keepdims=True)) + 1e-6

    w_cols = []
    for t in range(TOPK):
        it = idx_cols[t][:, None]
        tx = (it % n_grid).astype(f32)
        ty = (it // n_grid).astype(f32)
        g = jnp.exp(-0.5 * (((tx - ix) / sx) ** 2 + ((ty - iy) / sy) ** 2))
        w_cols.append(g / denom)

    idx_ref[...] = jnp.concatenate([c[:, None] for c in idx_cols], axis=1)
    gwt_ref[...] = jnp.concatenate(w_cols, axis=1)


def _attn_kernel(idx_ref, gwt_ref, q_ref, vdw_ref, *rest, scale, rb):
    f32 = jnp.float32
    kv_refs = rest[:rb * TOPK]
    pw_ref, pb_ref = rest[rb * TOPK], rest[rb * TOPK + 1]
    o_ref = rest[rb * TOPK + 2]
    i = pl.program_id(0)
    bf16 = jnp.bfloat16
    C = q_ref.shape[-1]
    lane = jax.lax.broadcasted_iota(jnp.int32, (1, TP), 1)
    colbias = jnp.where(lane % 8 == WS, jnp.float32(-jnp.inf), 0.0)
    outs = []
    for r in range(rb):
        q = q_ref[r]
        base = (i * rb + r) * TOPK
        ws_ = [gwt_ref[base + j] for j in range(TOPK)]
        sjs = [
            jax.lax.dot_general(
                q, kv_refs[r * TOPK + j][0, 0], (((1,), (1,)), ((), ())),
                preferred_element_type=f32) * (ws_[j] * scale) + colbias
            for j in range(TOPK)
        ]
        m = jnp.maximum(
            jnp.maximum(jnp.max(sjs[0], axis=1, keepdims=True),
                        jnp.max(sjs[1], axis=1, keepdims=True)),
            jnp.maximum(jnp.max(sjs[2], axis=1, keepdims=True),
                        jnp.max(sjs[3], axis=1, keepdims=True)))
        es = [jnp.exp(sj - m) for sj in sjs]
        denom = ((jnp.sum(es[0], axis=1, keepdims=True)
                  + jnp.sum(es[1], axis=1, keepdims=True))
                 + (jnp.sum(es[2], axis=1, keepdims=True)
                    + jnp.sum(es[3], axis=1, keepdims=True)))
        a01 = jax.lax.dot_general(
            (es[0] * ws_[0]).astype(bf16), kv_refs[r * TOPK][0, 1],
            (((1,), (0,)), ((), ())), preferred_element_type=f32)
        a01 += jax.lax.dot_general(
            (es[1] * ws_[1]).astype(bf16), kv_refs[r * TOPK + 1][0, 1],
            (((1,), (0,)), ((), ())), preferred_element_type=f32)
        a23 = jax.lax.dot_general(
            (es[2] * ws_[2]).astype(bf16), kv_refs[r * TOPK + 2][0, 1],
            (((1,), (0,)), ((), ())), preferred_element_type=f32)
        a23 += jax.lax.dot_general(
            (es[3] * ws_[3]).astype(bf16), kv_refs[r * TOPK + 3][0, 1],
            (((1,), (0,)), ((), ())), preferred_element_type=f32)
        outs.append((a01 + a23) * (1.0 / denom) + vdw_ref[r].astype(f32))
    o = jnp.concatenate(outs, axis=0)
    o = jnp.dot(o, pw_ref[...], preferred_element_type=f32) + pb_ref[...]
    o = o.reshape(rb, WS, 8, C).transpose(1, 0, 2, 3)[:, :, :WS, :]
    o_ref[...] = o.reshape(WS, rb * WS, C).transpose(0, 2, 1)


def kernel(x, qkv_w, qkv_b, gp_w1, gp_b1, gp_w2, gp_b2, dw_w, dw_b,
           proj_w, proj_b):
    B, H, W, C = x.shape
    ws = WS
    nH, nW = H // ws, W // ws
    S = nH * nW
    scale = float(C // NUM_HEADS) ** (-0.5)
    f32 = jnp.float32

    x3 = jnp.transpose(x[0], (0, 2, 1))
    wq = qkv_w[0:C].T
    wk = qkv_w[C:2 * C].T
    wv = qkv_w[2 * C:3 * C].T
    bq = qkv_b[0:C][None]
    bk = qkv_b[C:2 * C][None]
    bv = qkv_b[2 * C:3 * C][None]
    dw2 = dw_w[:, 0].transpose(1, 2, 0)
    dwb2 = dw_b[None]

    g1 = gp_w1.T
    gb1 = gp_b1[None]
    g2 = gp_w2.T
    gb2 = gp_b2[None]
    full2 = lambda a: pl.BlockSpec(a.shape, lambda i: (0,) * a.ndim)
    q, kv, vdw, xr, idx, gwt = pl.pallas_call(
        functools.partial(_qkv_kernel, scale=scale, n_grid=nW, n_steps=nH),
        grid=(nH,),
        in_specs=[
            pl.BlockSpec((ws, C, W), lambda i: (i, 0, 0)),
            full2(wq), full2(wk), full2(wv), full2(bq), full2(bk), full2(bv),
            full2(dw2), full2(dwb2),
            full2(g1), full2(gb1), full2(g2), full2(gb2),
        ],
        out_specs=[
            pl.BlockSpec((nW, TP, C), lambda i: (i, 0, 0)),
            pl.BlockSpec((nW, 2, TP, C), lambda i: (i, 0, 0, 0)),
            pl.BlockSpec((nW, TP, C), lambda i: (i, 0, 0)),
            pl.BlockSpec((S, C), lambda i: (0, 0)),
            pl.BlockSpec((S, TOPK), lambda i: (0, 0)),
            pl.BlockSpec((S, TOPK), lambda i: (0, 0)),
        ],
        out_shape=[
            jax.ShapeDtypeStruct((S, TP, C), jnp.bfloat16),
            jax.ShapeDtypeStruct((S, 2, TP, C), jnp.bfloat16),
            jax.ShapeDtypeStruct((S, TP, C), jnp.bfloat16),
            jax.ShapeDtypeStruct((S, C), f32),
            jax.ShapeDtypeStruct((S, TOPK), jnp.int32),
            jax.ShapeDtypeStruct((S, TOPK), f32),
        ],
    )(x3, wq, wk, wv, bq, bk, bv, dw2, dwb2, g1, gb1, g2, gb2)

    pw = proj_w.T
    pb = proj_b[None]

    RB = nW
    own = pl.BlockSpec((RB, TP, C), lambda i, idx_r, gwt_r: (i, 0, 0))

    def gspec(r, j):
        return pl.BlockSpec(
            (1, 2, TP, C),
            lambda i, idx_r, gwt_r: (idx_r[(i * RB + r) * TOPK + j], 0, 0, 0))

    fullp = lambda a: pl.BlockSpec(
        a.shape, lambda i, idx_r, gwt_r: (0,) * a.ndim)

    out_t = pl.pallas_call(
        functools.partial(_attn_kernel, scale=scale, rb=RB),
        grid_spec=pltpu.PrefetchScalarGridSpec(
            num_scalar_prefetch=2,
            grid=(S // RB,),
            in_specs=[own, own]
            + [gspec(r, j) for r in range(RB) for j in range(TOPK)]
            + [fullp(pw), fullp(pb)],
            out_specs=pl.BlockSpec(
                (ws, C, RB * ws),
                lambda i, idx_r, gwt_r: (i, 0, 0)),
        ),
        out_shape=jax.ShapeDtypeStruct((H, C, W), f32),
    )(idx.reshape(-1), gwt.reshape(-1), q, vdw,
      *([kv] * (RB * TOPK)), pw, pb)

    return jnp.transpose(out_t, (0, 2, 1)).reshape(1, H, W, C)
